# Initial kernel scaffold; baseline (speedup 1.0000x reference)
import jax
import jax.numpy as jnp
from jax.experimental import pallas as pl


def kernel(x, g_edge_index, lg_edge_index, index01, W1, b1, W2, b2, Wl, bl):
    raise NotImplementedError("write your pallas kernel here")


# speedup vs baseline: 1.0182x; 1.0182x over previous
import functools

import jax
import jax.numpy as jnp
from jax import lax
from jax.experimental import pallas as pl
from jax.experimental.pallas import tpu as pltpu
from jax.experimental.pallas import tpu_sc as plsc

N = 10000
E = 160000
D = 128
H = 128

NC, NS = 2, 16
NW = NC * NS


def _sc_gather(table, idx, chunk=200):
    B = idx.shape[0]
    assert B % (8 * NW) == 0
    b_per_w = B // NW
    assert b_per_w % chunk == 0 and chunk % 8 == 0
    n_chunks = b_per_w // chunk
    mesh = plsc.VectorSubcoreMesh(core_axis_name="c", subcore_axis_name="s")

    @functools.partial(
        pl.kernel,
        out_type=jax.ShapeDtypeStruct((B, table.shape[1]), table.dtype),
        mesh=mesh,
        scratch_types=[
            pltpu.VMEM((b_per_w,), jnp.int32),
            pltpu.VMEM((chunk, table.shape[1]), table.dtype),
            pltpu.SemaphoreType.DMA,
        ],
    )
    def k(table_hbm, idx_hbm, out_hbm, idx_v, rows_v, sem):
        wid = lax.axis_index("s") * NC + lax.axis_index("c")
        base = wid * b_per_w
        pltpu.sync_copy(idx_hbm.at[pl.ds(base, b_per_w)], idx_v)
        for c in range(n_chunks):
            pltpu.async_copy(
                table_hbm.at[idx_v.at[pl.ds(c * chunk, chunk)]], rows_v, sem
            ).wait()
            pltpu.sync_copy(rows_v, out_hbm.at[pl.ds(base + c * chunk, chunk)])

    return k(table, idx)


def _gcn_conv_jnp(h, edge_index, W, b):
    n = h.shape[0]
    loop = jnp.arange(n, dtype=edge_index.dtype)
    src = jnp.concatenate([edge_index[0], loop])
    dst = jnp.concatenate([edge_index[1], loop])
    ones = jnp.ones(src.shape[0], dtype=h.dtype)
    deg = jnp.zeros((n,), dtype=h.dtype).at[dst].add(ones)
    dinv = jnp.where(deg > 0, deg ** -0.5, 0.0)
    hw = h @ W
    norm = dinv[src] * dinv[dst]
    msg = hw[src] * norm[:, None]
    out = jnp.zeros((n, W.shape[1]), dtype=h.dtype).at[dst].add(msg)
    return out + b


def kernel(x, g_edge_index, lg_edge_index, index01, W1, b1, W2, b2, Wl, bl):
    xw_a = x @ W1[:D]
    xw_b = x @ W1[D:]
    table = jnp.concatenate([xw_a, xw_b], axis=0)
    idx2 = jnp.concatenate([g_edge_index[0], g_edge_index[1] + N])
    rows = _sc_gather(table, idx2)
    hw1 = rows[:E] + rows[E:]

    loop = jnp.arange(E, dtype=lg_edge_index.dtype)
    src = jnp.concatenate([lg_edge_index[0], loop])
    dst = jnp.concatenate([lg_edge_index[1], loop])
    ones = jnp.ones(src.shape[0], dtype=jnp.float32)
    deg = jnp.zeros((E,), dtype=jnp.float32).at[dst].add(ones)
    dinv = jnp.where(deg > 0, deg ** -0.5, 0.0)
    norm = dinv[src] * dinv[dst]
    msg = hw1[src] * norm[:, None]
    out1 = jnp.zeros((E, H), dtype=jnp.float32).at[dst].add(msg)
    h = jax.nn.relu(out1 + b1)

    h = jax.nn.relu(_gcn_conv_jnp(h, lg_edge_index, W2, b2))
    sel = h[index01][None, :, :]
    return jax.nn.sigmoid(sel @ Wl + bl)


# speedup vs baseline: 1.4404x; 1.4146x over previous
import functools

import jax
import jax.numpy as jnp
from jax import lax
from jax.experimental import pallas as pl
from jax.experimental.pallas import tpu as pltpu
from jax.experimental.pallas import tpu_sc as plsc

N = 10000
E = 160000
ELG = 1280000
D = 128
H = 128

NC, NS = 2, 16
NW = NC * NS

R = 8192
NPASS = (E + 2 * R - 1) // (2 * R)
EPW = ELG // NS
SCAN = 8000
NSCAN = EPW // SCAN
VEC = 16
CHUNK = 128
STAGE_ROWS = SCAN // CHUNK + 1
RPT = R // NS
LAST_RPT = (E - (2 * NPASS - 1) * R) // NS

_SC_PARAMS = pltpu.CompilerParams(needs_layout_passes=False)


def _sc_gather(table, idx, chunk=200):
    B = idx.shape[0]
    assert B % (8 * NW) == 0
    b_per_w = B // NW
    assert b_per_w % chunk == 0 and chunk % 8 == 0
    n_chunks = b_per_w // chunk
    mesh = plsc.VectorSubcoreMesh(core_axis_name="c", subcore_axis_name="s")

    @functools.partial(
        pl.kernel,
        out_type=jax.ShapeDtypeStruct((B, table.shape[1]), table.dtype),
        mesh=mesh,
        compiler_params=_SC_PARAMS,
        scratch_types=[
            pltpu.VMEM((b_per_w,), jnp.int32),
            pltpu.VMEM((chunk, table.shape[1]), table.dtype),
            pltpu.SemaphoreType.DMA,
        ],
    )
    def k(table_hbm, idx_hbm, out_hbm, idx_v, rows_v, sem):
        wid = lax.axis_index("s") * NC + lax.axis_index("c")
        base = wid * b_per_w
        pltpu.sync_copy(idx_hbm.at[pl.ds(base, b_per_w)], idx_v)
        for cc in range(n_chunks):
            pltpu.async_copy(
                table_hbm.at[idx_v.at[pl.ds(cc * chunk, chunk)]], rows_v, sem
            ).wait()
            pltpu.sync_copy(rows_v, out_hbm.at[pl.ds(base + cc * chunk, chunk)])

    return k(table, idx)


def _sc_segsum(g, src, dst):
    mesh = plsc.VectorSubcoreMesh(core_axis_name="c", subcore_axis_name="s")

    @functools.partial(
        pl.kernel,
        out_type=jax.ShapeDtypeStruct((E, H), jnp.float32),
        mesh=mesh,
        compiler_params=_SC_PARAMS,
        scratch_types=[
            pltpu.VMEM((SCAN,), jnp.int32),
            pltpu.VMEM((SCAN,), jnp.int32),
            pltpu.VMEM((STAGE_ROWS, CHUNK), jnp.int32),
            pltpu.VMEM((STAGE_ROWS, CHUNK), jnp.int32),
            pltpu.VMEM((CHUNK, H), jnp.float32),
            pltpu.VMEM((128, H), jnp.float32),
            pltpu.VMEM_SHARED((R + 8, H), jnp.float32),
            pltpu.SemaphoreType.DMA,
        ],
    )
    def k(g_hbm, src_hbm, dst_hbm, zeros_hbm, out_hbm,
          src_scan, dst_scan, src_stage, dst_stage, rows, zblk, acc, sem):
        c = lax.axis_index("c")
        s = lax.axis_index("s")
        iota = lax.iota(jnp.int32, VEC)

        pltpu.sync_copy(zeros_hbm, zblk)

        for p in range(NPASS):
            lo = (2 * p + c) * R
            for z in range(RPT // 128):
                pltpu.sync_copy(zblk, acc.at[pl.ds(s * RPT + z * 128, 128)])
            plsc.subcore_barrier()

            def scan_chunk(kk, _):
                base = s * EPW + kk * SCAN
                pltpu.sync_copy(src_hbm.at[pl.ds(base, SCAN)], src_scan)
                pltpu.sync_copy(dst_hbm.at[pl.ds(base, SCAN)], dst_scan)

                def vec_iter(i, cnt):
                    dv = dst_scan[pl.ds(i * VEC, VEC)]
                    loc = dv - lo
                    m = (loc >= 0) & (loc < R)
                    csum = plsc.cumsum(m.astype(jnp.int32))
                    pos = cnt + csum - 1
                    prow = lax.shift_right_logical(pos, 7)
                    pcol = pos & (CHUNK - 1)
                    sv = src_scan[pl.ds(i * VEC, VEC)]
                    plsc.store_scatter(src_stage, [prow, pcol], sv, mask=m)
                    plsc.store_scatter(dst_stage, [prow, pcol], loc, mask=m)
                    return cnt + jnp.sum(m.astype(jnp.int32))

                cnt = lax.fori_loop(0, SCAN // VEC, vec_iter, jnp.int32(0))

                rup = lax.shift_left(
                    lax.shift_right_logical(cnt + CHUNK - 1, 7), 7)
                dummy_dst = jnp.full((VEC,), R, jnp.int32)
                dummy_src = jnp.zeros((VEC,), jnp.int32)
                for j in range(8):
                    pos = cnt + j * VEC + iota
                    mm = pos < rup
                    prow = lax.shift_right_logical(pos, 7)
                    pcol = pos & (CHUNK - 1)
                    plsc.store_scatter(src_stage, [prow, pcol], dummy_src,
                                       mask=mm)
                    plsc.store_scatter(dst_stage, [prow, pcol], dummy_dst,
                                       mask=mm)

                def gather_add(j, _):
                    pltpu.async_copy(g_hbm.at[src_stage.at[j]], rows,
                                     sem).wait()
                    pltpu.sync_copy(rows, acc.at[dst_stage.at[j]], add=True)
                    return 0

                lax.fori_loop(0, lax.shift_right_logical(rup, 7),
                              gather_add, 0)
                return 0

            lax.fori_loop(0, NSCAN, scan_chunk, 0)
            plsc.subcore_barrier()

            def flush(n_rows):
                pltpu.sync_copy(acc.at[pl.ds(s * n_rows, n_rows)],
                                out_hbm.at[pl.ds(lo + s * n_rows, n_rows)])

            if p < NPASS - 1:
                flush(RPT)
            else:
                @pl.when(c == 0)
                def _():
                    flush(RPT)

                @pl.when(c == 1)
                def _():
                    flush(LAST_RPT)
            plsc.subcore_barrier()

    return k(g, src, dst, jnp.zeros((128, H), jnp.float32))


def kernel(x, g_edge_index, lg_edge_index, index01, W1, b1, W2, b2, Wl, bl):
    xw_a = x @ W1[:D]
    xw_b = x @ W1[D:]
    table = jnp.concatenate([xw_a, xw_b], axis=0)
    idx2 = jnp.concatenate([g_edge_index[0], g_edge_index[1] + N])
    rows = _sc_gather(table, idx2)
    hw1 = rows[:E] + rows[E:]

    lg_src = lg_edge_index[0]
    lg_dst = lg_edge_index[1]
    deg = jnp.ones((E,), jnp.float32).at[lg_dst].add(
        jnp.ones((ELG,), jnp.float32))
    dinv = deg ** -0.5

    g1 = dinv[:, None] * hw1
    S1 = _sc_segsum(g1, lg_src, lg_dst)
    h2 = jax.nn.relu(dinv[:, None] * (S1 + g1) + b1)

    g2 = dinv[:, None] * (h2 @ W2)
    S2 = _sc_segsum(g2, lg_src, lg_dst)
    h3 = jax.nn.relu(dinv[:, None] * (S2 + g2) + b2)

    sel = h3[index01][None, :, :]
    return jax.nn.sigmoid(sel @ Wl + bl)
